# baseline (speedup 1.0000x reference)
import functools

import jax
import jax.numpy as jnp
from jax.experimental import pallas as pl
from jax.experimental.pallas import tpu as pltpu

_L = 4
_R = 4
_CS = 24


def _backbone_kernel(x_ref, ring_ref,
                     w1_ref, b1_ref, w2_ref, b2_ref, w3_ref, b3_ref,
                     w4_ref, b4_ref, w5_ref, b5_ref,
                     o_ref, zp_ref, ysc_ref, *, HWp, Wp, PAD):
    BLK = 2 * PAD + HWp
    M = _R * BLK
    ring = ring_ref[...]

    z8 = jnp.zeros((PAD + 8, 128), zp_ref.dtype)
    zp_ref[0:PAD + 8, :] = z8
    zp_ref[8 + M - PAD:16 + M, :] = jnp.zeros((PAD + 8, 128), zp_ref.dtype)
    zpad = jnp.zeros((2 * PAD, 128), zp_ref.dtype)
    for r in range(_R - 1):
        lo = 8 + r * BLK + PAD + HWp
        zp_ref[lo:lo + 2 * PAD, :] = zpad

    for r in range(_R):
        lo = 8 + r * BLK + PAD
        zp_ref[lo:lo + HWp, :] = x_ref[0, r]

    def conv3x3_relu(w_ref, b_ref, store_out):
        ysc_ref[...] = (
            jnp.dot(zp_ref[7:7 + M, :], w_ref[0:128, :],
                    preferred_element_type=jnp.float32)
            + jnp.dot(zp_ref[8:8 + M, :], w_ref[128:256, :],
                      preferred_element_type=jnp.float32)
            + jnp.dot(zp_ref[9:9 + M, :], w_ref[256:384, :],
                      preferred_element_type=jnp.float32))
        b = b_ref[...]
        for r in range(_R):
            b0 = r * BLK
            acc = (ysc_ref[b0 + PAD - Wp:b0 + PAD - Wp + HWp, 0:128]
                   + ysc_ref[b0 + PAD:b0 + PAD + HWp, 128:256]
                   + ysc_ref[b0 + PAD + Wp:b0 + PAD + Wp + HWp, 256:384])
            h = jnp.maximum(acc + b, 0.0) * ring
            if store_out:
                for i in range(_L):
                    o_ref[r * _L + i] = h[:, i * _CS:(i + 1) * _CS].astype(
                        o_ref.dtype)
            else:
                lo = 8 + b0 + PAD
                zp_ref[lo:lo + HWp, :] = h.astype(zp_ref.dtype)

    conv3x3_relu(w1_ref, b1_ref, False)
    conv3x3_relu(w2_ref, b2_ref, False)
    conv3x3_relu(w3_ref, b3_ref, False)
    conv3x3_relu(w4_ref, b4_ref, False)
    conv3x3_relu(w5_ref, b5_ref, True)


def _fc_kernel(x_ref, w_ref, o_ref):
    o_ref[0] = jnp.dot(x_ref[...], w_ref[...].astype(jnp.bfloat16),
                       preferred_element_type=jnp.float32)


def _const_spec(arr):
    nd = arr.ndim
    return pl.BlockSpec(arr.shape, lambda b, _nd=nd: (0,) * _nd)


def _quad_weights(w):
    cout = w.shape[-1]
    cin = w.shape[1] // 3
    wk = w.reshape(3, 3, cin, cout)
    wt = jnp.transpose(wk, (1, 2, 0, 3))
    z = jnp.zeros((3, 128, 3, 128), jnp.float32)
    for i in range(_L):
        z = z.at[:, i * _CS:i * _CS + cin, :, i * _CS:i * _CS + cout].set(wt)
    return z.reshape(384, 384).astype(jnp.bfloat16)


def _quad_bias(b):
    cout = b.shape[-1]
    bq = jnp.zeros((1, 128), jnp.float32)
    for i in range(_L):
        bq = bq.at[:, i * _CS:i * _CS + cout].set(b)
    return bq


def kernel(x_nchw, w1, b1, w2, b2, w3, b3, w4, b4, w5, b5, wfc, bfc, ring):
    N, Cin, Himg, Wimg = x_nchw.shape
    HWp = Himg * Wimg
    PAD = ((Wimg + 1 + 7) // 8) * 8
    BLK = 2 * PAD + HWp
    C5 = b5.shape[-1]
    out_dim = bfc.shape[-1]
    GP = _L * _R
    NG = N // GP

    x_emb = jnp.transpose(x_nchw, (0, 2, 3, 1)).reshape(N, HWp, Cin)
    xq = jnp.pad(x_emb.astype(jnp.bfloat16), ((0, 0), (0, 0), (0, _CS - Cin)))
    xq = xq.reshape(NG, _R, _L, HWp, _CS).transpose(0, 1, 3, 2, 4)
    xq = jnp.pad(xq.reshape(NG, _R, HWp, _L * _CS),
                 ((0, 0), (0, 0), (0, 0), (0, 128 - _L * _CS)))

    weight_args = [_quad_weights(w1), _quad_bias(b1),
                   _quad_weights(w2), _quad_bias(b2),
                   _quad_weights(w3), _quad_bias(b3),
                   _quad_weights(w4), _quad_bias(b4),
                   _quad_weights(w5), _quad_bias(b5)]

    feat = pl.pallas_call(
        functools.partial(_backbone_kernel, HWp=HWp, Wp=Wimg, PAD=PAD),
        out_shape=jax.ShapeDtypeStruct((N, HWp, C5), jnp.bfloat16),
        grid=(NG,),
        in_specs=([pl.BlockSpec((1, _R, HWp, 128),
                                lambda b: (b, 0, 0, 0)),
                   _const_spec(ring)]
                  + [_const_spec(a) for a in weight_args]),
        out_specs=pl.BlockSpec((GP, HWp, C5), lambda b: (b, 0, 0)),
        scratch_shapes=[pltpu.VMEM((16 + _R * BLK, 128), jnp.bfloat16),
                        pltpu.VMEM((_R * BLK, 384), jnp.float32)],
        compiler_params=pltpu.CompilerParams(
            dimension_semantics=("parallel",)),
    )(xq, ring, *weight_args)

    flat = feat.reshape(N, HWp * C5)
    K = HWp * C5
    KS = 2 if K % 2 == 0 else 1
    Kh = K // KS
    partial = pl.pallas_call(
        _fc_kernel,
        out_shape=jax.ShapeDtypeStruct((KS, N, out_dim), jnp.float32),
        grid=(KS,),
        in_specs=[pl.BlockSpec((N, Kh), lambda k: (0, k)),
                  pl.BlockSpec((Kh, out_dim), lambda k: (k, 0))],
        out_specs=pl.BlockSpec((1, N, out_dim), lambda k: (k, 0, 0)),
        compiler_params=pltpu.CompilerParams(
            dimension_semantics=("parallel",)),
    )(flat, wfc)
    return partial.sum(axis=0) + bfc


# speedup vs baseline: 1.5213x; 1.5213x over previous
import functools

import jax
import jax.numpy as jnp
from jax.experimental import pallas as pl
from jax.experimental.pallas import tpu as pltpu

_L = 4
_R = 4
_CS = 24


def _backbone_kernel(x_ref, ring_ref,
                     w1_ref, b1_ref, w2_ref, b2_ref, w3_ref, b3_ref,
                     w4_ref, b4_ref, w5_ref, b5_ref,
                     o_ref, zp_ref, ysc_ref, *, HWp, Wp, PAD):
    BLK = 2 * PAD + HWp
    M = _R * BLK
    ring = ring_ref[...]

    z8 = jnp.zeros((PAD + 8, 128), zp_ref.dtype)
    zp_ref[0:PAD + 8, :] = z8
    zp_ref[8 + M - PAD:16 + M, :] = jnp.zeros((PAD + 8, 128), zp_ref.dtype)
    zpad = jnp.zeros((2 * PAD, 128), zp_ref.dtype)
    for r in range(_R - 1):
        lo = 8 + r * BLK + PAD + HWp
        zp_ref[lo:lo + 2 * PAD, :] = zpad

    for r in range(_R):
        lo = 8 + r * BLK + PAD
        zp_ref[lo:lo + HWp, :] = x_ref[0, r]

    def conv3x3_relu(w_ref, b_ref, store_out, first=False):
        if first:
            ysc_ref[...] = jnp.dot(
                zp_ref[8:8 + M, :], w_ref[...],
                preferred_element_type=jnp.float32)
        else:
            patches = jnp.concatenate(
                [zp_ref[7:7 + M, :],
                 zp_ref[8:8 + M, :],
                 zp_ref[9:9 + M, :]], axis=-1)
            ysc_ref[...] = jnp.dot(
                patches, w_ref[...], preferred_element_type=jnp.float32)
        b = b_ref[...]
        for r in range(_R):
            b0 = r * BLK
            acc = (ysc_ref[b0 + PAD - Wp:b0 + PAD - Wp + HWp, 0:128]
                   + ysc_ref[b0 + PAD:b0 + PAD + HWp, 128:256]
                   + ysc_ref[b0 + PAD + Wp:b0 + PAD + Wp + HWp, 256:384])
            h = jnp.maximum(acc + b, 0.0) * ring
            if store_out:
                for i in range(_L):
                    o_ref[r * _L + i] = h[:, i * _CS:(i + 1) * _CS].astype(
                        o_ref.dtype)
            else:
                lo = 8 + b0 + PAD
                zp_ref[lo:lo + HWp, :] = h.astype(zp_ref.dtype)

    conv3x3_relu(w1_ref, b1_ref, False, first=True)
    conv3x3_relu(w2_ref, b2_ref, False)
    conv3x3_relu(w3_ref, b3_ref, False)
    conv3x3_relu(w4_ref, b4_ref, False)
    conv3x3_relu(w5_ref, b5_ref, True)


def _fc_kernel(x_ref, w_ref, o_ref):
    o_ref[0] = jnp.dot(x_ref[...], w_ref[...].astype(jnp.bfloat16),
                       preferred_element_type=jnp.float32)


def _const_spec(arr):
    nd = arr.ndim
    return pl.BlockSpec(arr.shape, lambda b, _nd=nd: (0,) * _nd)


def _quad_weights(w):
    cout = w.shape[-1]
    cin = w.shape[1] // 3
    wk = w.reshape(3, 3, cin, cout)
    wt = jnp.transpose(wk, (1, 2, 0, 3))
    z = jnp.zeros((3, 128, 3, 128), jnp.float32)
    for i in range(_L):
        z = z.at[:, i * _CS:i * _CS + cin, :, i * _CS:i * _CS + cout].set(wt)
    return z.reshape(384, 384).astype(jnp.bfloat16)


def _quad_weights1(w, Cin):
    cout = w.shape[-1]
    wk = w.reshape(3, 3, Cin, cout)
    wt = jnp.transpose(wk, (1, 2, 0, 3))
    wt9 = wt.reshape(3 * Cin, 3, cout)
    z = jnp.zeros((128, 3, 128), jnp.float32)
    for i in range(_L):
        z = z.at[i * 3 * Cin:(i + 1) * 3 * Cin, :,
                 i * _CS:i * _CS + cout].set(wt9)
    return z.reshape(128, 384).astype(jnp.bfloat16)


def _quad_bias(b):
    cout = b.shape[-1]
    bq = jnp.zeros((1, 128), jnp.float32)
    for i in range(_L):
        bq = bq.at[:, i * _CS:i * _CS + cout].set(b)
    return bq


def kernel(x_nchw, w1, b1, w2, b2, w3, b3, w4, b4, w5, b5, wfc, bfc, ring):
    N, Cin, Himg, Wimg = x_nchw.shape
    HWp = Himg * Wimg
    PAD = ((Wimg + 1 + 7) // 8) * 8
    BLK = 2 * PAD + HWp
    C5 = b5.shape[-1]
    out_dim = bfc.shape[-1]
    GP = _L * _R
    NG = N // GP

    x_emb = jnp.transpose(x_nchw, (0, 2, 3, 1)).reshape(N, HWp, Cin)
    x_emb = x_emb.astype(jnp.bfloat16)
    sh_m = jnp.pad(x_emb, ((0, 0), (1, 0), (0, 0)))[:, :HWp, :]
    sh_p = jnp.pad(x_emb, ((0, 0), (0, 1), (0, 0)))[:, 1:, :]
    xs = jnp.stack([sh_m, x_emb, sh_p], axis=2).reshape(N, HWp, 3 * Cin)
    xq = xs.reshape(NG, _R, _L, HWp, 3 * Cin).transpose(0, 1, 3, 2, 4)
    xq = jnp.pad(xq.reshape(NG, _R, HWp, _L * 3 * Cin),
                 ((0, 0), (0, 0), (0, 0), (0, 128 - _L * 3 * Cin)))

    weight_args = [_quad_weights1(w1, Cin), _quad_bias(b1),
                   _quad_weights(w2), _quad_bias(b2),
                   _quad_weights(w3), _quad_bias(b3),
                   _quad_weights(w4), _quad_bias(b4),
                   _quad_weights(w5), _quad_bias(b5)]

    feat = pl.pallas_call(
        functools.partial(_backbone_kernel, HWp=HWp, Wp=Wimg, PAD=PAD),
        out_shape=jax.ShapeDtypeStruct((N, HWp, C5), jnp.bfloat16),
        grid=(NG,),
        in_specs=([pl.BlockSpec((1, _R, HWp, 128),
                                lambda b: (b, 0, 0, 0)),
                   _const_spec(ring)]
                  + [_const_spec(a) for a in weight_args]),
        out_specs=pl.BlockSpec((GP, HWp, C5), lambda b: (b, 0, 0)),
        scratch_shapes=[pltpu.VMEM((16 + _R * BLK, 128), jnp.bfloat16),
                        pltpu.VMEM((_R * BLK, 384), jnp.float32)],
        compiler_params=pltpu.CompilerParams(
            dimension_semantics=("parallel",)),
    )(xq, ring, *weight_args)

    flat = feat.reshape(N, HWp * C5)
    K = HWp * C5
    KS = 2 if K % 2 == 0 else 1
    Kh = K // KS
    partial = pl.pallas_call(
        _fc_kernel,
        out_shape=jax.ShapeDtypeStruct((KS, N, out_dim), jnp.float32),
        grid=(KS,),
        in_specs=[pl.BlockSpec((N, Kh), lambda k: (0, k)),
                  pl.BlockSpec((Kh, out_dim), lambda k: (k, 0))],
        out_specs=pl.BlockSpec((1, N, out_dim), lambda k: (k, 0, 0)),
        compiler_params=pltpu.CompilerParams(
            dimension_semantics=("parallel",)),
    )(flat, wfc)
    return partial.sum(axis=0) + bfc


# speedup vs baseline: 1.7039x; 1.1200x over previous
import functools

import jax
import jax.numpy as jnp
from jax.experimental import pallas as pl
from jax.experimental.pallas import tpu as pltpu

_L = 4
_R = 4
_CS = 24


def _backbone_kernel(x_ref, ring_ref,
                     w1_ref, b1_ref, w2_ref, b2_ref, w3_ref, b3_ref,
                     w4_ref, b4_ref, w5_ref, b5_ref,
                     o_ref, zp_ref, ysc_ref, *, HWp, Wp, PAD):
    BLK = 2 * PAD + HWp
    M = _R * BLK
    ring = ring_ref[...]

    z8 = jnp.zeros((PAD + 8, 128), zp_ref.dtype)
    zp_ref[0:PAD + 8, :] = z8
    zp_ref[8 + M - PAD:16 + M, :] = jnp.zeros((PAD + 8, 128), zp_ref.dtype)
    zpad = jnp.zeros((2 * PAD, 128), zp_ref.dtype)
    for r in range(_R - 1):
        lo = 8 + r * BLK + PAD + HWp
        zp_ref[lo:lo + 2 * PAD, :] = zpad

    for r in range(_R):
        lo = 8 + r * BLK + PAD
        zp_ref[lo:lo + HWp, :] = x_ref[0, r]

    def conv3x3_relu(w_ref, b_ref, store_out, first=False):
        if first:
            ysc_ref[...] = jnp.dot(
                zp_ref[8:8 + M, :], w_ref[...],
                preferred_element_type=jnp.float32)
        else:
            patches = jnp.concatenate(
                [zp_ref[7:7 + M, :],
                 zp_ref[8:8 + M, :],
                 zp_ref[9:9 + M, :]], axis=-1)
            ysc_ref[...] = jnp.dot(
                patches, w_ref[...], preferred_element_type=jnp.float32)
        b = b_ref[...]
        for r in range(_R):
            b0 = r * BLK
            acc = (ysc_ref[b0 + PAD - Wp:b0 + PAD - Wp + HWp, 0:128]
                   + ysc_ref[b0 + PAD:b0 + PAD + HWp, 128:256]
                   + ysc_ref[b0 + PAD + Wp:b0 + PAD + Wp + HWp, 256:384])
            h = jnp.maximum(acc + b, 0.0) * ring
            if store_out:
                for i in range(_L):
                    o_ref[r * _L + i] = h[:, i * _CS:(i + 1) * _CS].astype(
                        o_ref.dtype)
            else:
                lo = 8 + b0 + PAD
                zp_ref[lo:lo + HWp, :] = h.astype(zp_ref.dtype)

    conv3x3_relu(w1_ref, b1_ref, False, first=True)
    conv3x3_relu(w2_ref, b2_ref, False)
    conv3x3_relu(w3_ref, b3_ref, False)
    conv3x3_relu(w4_ref, b4_ref, False)
    conv3x3_relu(w5_ref, b5_ref, True)


def _fc_kernel(x_ref, w_ref, o_ref):
    n = x_ref.shape[0]
    kc = x_ref.shape[1] * x_ref.shape[2]
    x2 = x_ref[...].reshape(n, kc)
    w2 = w_ref[...].reshape(kc, w_ref.shape[2]).astype(jnp.bfloat16)
    y = jnp.dot(x2, w2, preferred_element_type=jnp.float32)
    j = pl.program_id(1)

    @pl.when(j == 0)
    def _():
        o_ref[0] = y

    @pl.when(j > 0)
    def _():
        o_ref[0] = o_ref[0] + y


def _const_spec(arr):
    nd = arr.ndim
    return pl.BlockSpec(arr.shape, lambda b, _nd=nd: (0,) * _nd)


def _quad_weights(w):
    cout = w.shape[-1]
    cin = w.shape[1] // 3
    wk = w.reshape(3, 3, cin, cout)
    wt = jnp.transpose(wk, (1, 2, 0, 3))
    z = jnp.zeros((3, 128, 3, 128), jnp.float32)
    for i in range(_L):
        z = z.at[:, i * _CS:i * _CS + cin, :, i * _CS:i * _CS + cout].set(wt)
    return z.reshape(384, 384).astype(jnp.bfloat16)


def _quad_weights1(w, Cin):
    cout = w.shape[-1]
    wk = w.reshape(3, 3, Cin, cout)
    wt = jnp.transpose(wk, (1, 2, 0, 3))
    wt9 = wt.reshape(3 * Cin, 3, cout)
    z = jnp.zeros((128, 3, 128), jnp.float32)
    for i in range(_L):
        z = z.at[i * 3 * Cin:(i + 1) * 3 * Cin, :,
                 i * _CS:i * _CS + cout].set(wt9)
    return z.reshape(128, 384).astype(jnp.bfloat16)


def _quad_bias(b):
    cout = b.shape[-1]
    bq = jnp.zeros((1, 128), jnp.float32)
    for i in range(_L):
        bq = bq.at[:, i * _CS:i * _CS + cout].set(b)
    return bq


def kernel(x_nchw, w1, b1, w2, b2, w3, b3, w4, b4, w5, b5, wfc, bfc, ring):
    N, Cin, Himg, Wimg = x_nchw.shape
    HWp = Himg * Wimg
    PAD = ((Wimg + 1 + 7) // 8) * 8
    BLK = 2 * PAD + HWp
    C5 = b5.shape[-1]
    out_dim = bfc.shape[-1]
    GP = _L * _R
    NG = N // GP

    x_emb = jnp.transpose(x_nchw, (0, 2, 3, 1)).reshape(N, HWp, Cin)
    x_emb = x_emb.astype(jnp.bfloat16)
    sh_m = jnp.pad(x_emb, ((0, 0), (1, 0), (0, 0)))[:, :HWp, :]
    sh_p = jnp.pad(x_emb, ((0, 0), (0, 1), (0, 0)))[:, 1:, :]
    xs = jnp.stack([sh_m, x_emb, sh_p], axis=2).reshape(N, HWp, 3 * Cin)
    xq = xs.reshape(NG, _R, _L, HWp, 3 * Cin).transpose(0, 1, 3, 2, 4)
    xq = jnp.pad(xq.reshape(NG, _R, HWp, _L * 3 * Cin),
                 ((0, 0), (0, 0), (0, 0), (0, 128 - _L * 3 * Cin)))

    weight_args = [_quad_weights1(w1, Cin), _quad_bias(b1),
                   _quad_weights(w2), _quad_bias(b2),
                   _quad_weights(w3), _quad_bias(b3),
                   _quad_weights(w4), _quad_bias(b4),
                   _quad_weights(w5), _quad_bias(b5)]

    feat = pl.pallas_call(
        functools.partial(_backbone_kernel, HWp=HWp, Wp=Wimg, PAD=PAD),
        out_shape=jax.ShapeDtypeStruct((N, HWp, C5), jnp.bfloat16),
        grid=(NG,),
        in_specs=([pl.BlockSpec((1, _R, HWp, 128),
                                lambda b: (b, 0, 0, 0)),
                   _const_spec(ring)]
                  + [_const_spec(a) for a in weight_args]),
        out_specs=pl.BlockSpec((GP, HWp, C5), lambda b: (b, 0, 0)),
        scratch_shapes=[pltpu.VMEM((16 + _R * BLK, 128), jnp.bfloat16),
                        pltpu.VMEM((_R * BLK, 384), jnp.float32)],
        compiler_params=pltpu.CompilerParams(
            dimension_semantics=("parallel",)),
    )(xq, ring, *weight_args)

    wfc3 = wfc.reshape(HWp, C5, out_dim)
    KS = 2
    KJ = max(d for d in range(1, 11) if (HWp // KS) % (8 * d) == 0)
    Hh = HWp // (KS * KJ)
    partial = pl.pallas_call(
        _fc_kernel,
        out_shape=jax.ShapeDtypeStruct((KS, N, out_dim), jnp.float32),
        grid=(KS, KJ),
        in_specs=[pl.BlockSpec((N, Hh, C5), lambda k, j: (0, k * KJ + j, 0)),
                  pl.BlockSpec((Hh, C5, out_dim),
                               lambda k, j: (k * KJ + j, 0, 0))],
        out_specs=pl.BlockSpec((1, N, out_dim), lambda k, j: (k, 0, 0)),
        compiler_params=pltpu.CompilerParams(
            dimension_semantics=("parallel", "arbitrary")),
    )(feat, wfc3)
    return partial.sum(axis=0) + bfc
